# Initial kernel scaffold; baseline (speedup 1.0000x reference)
import jax
import jax.numpy as jnp
from jax.experimental import pallas as pl


def kernel(x, codebooks):
    raise NotImplementedError("write your pallas kernel here")


# speedup vs baseline: 2.4394x; 2.4394x over previous
import functools

import jax
import jax.numpy as jnp
from jax.experimental import pallas as pl

B, T, INPUT_DIM = 16, 1024, 256
NUM_GROUPS = 8
CODES_PER_GROUP = 1024
GROUP_DIM = INPUT_DIM // NUM_GROUPS
N = B * T

ROWS_PER_BLOCK = 512


def _pq_body(x_ref, cbt_ref, cbn_ref, xn_ref, xq_ref, idx_ref, loss_ref):
    pid = pl.program_id(0)

    @pl.when(pid == 0)
    def _init():
        loss_ref[...] = jnp.zeros((1, 1), jnp.float32)

    acc = jnp.zeros((1, 1), jnp.float32)
    for g in range(NUM_GROUPS):
        xg = x_ref[:, g * GROUP_DIM:(g + 1) * GROUP_DIM]
        mm = jnp.dot(xg, cbt_ref[g], preferred_element_type=jnp.float32)
        d = (xn_ref[g, :][:, None] + cbn_ref[g, :][None, :]) - 2.0 * mm
        m = jnp.min(d, axis=1, keepdims=True)
        iota = jax.lax.broadcasted_iota(jnp.int32, d.shape, 1)
        idxv = jnp.min(jnp.where(d == m, iota, CODES_PER_GROUP), axis=1)
        idx_ref[g, :] = idxv
        onehot = (iota == idxv[:, None]).astype(jnp.float32)
        xq = jax.lax.dot_general(
            onehot, cbt_ref[g], (((1,), (1,)), ((), ())),
            preferred_element_type=jnp.float32)
        xq_ref[:, g * GROUP_DIM:(g + 1) * GROUP_DIM] = xg + (xq - xg)
        acc = acc + jnp.sum(m, axis=0, keepdims=True)

    loss_ref[...] += acc


@jax.jit
def kernel(x, codebooks):
    original_shape = x.shape
    if x.ndim == 2:
        x3 = x[:, None, :]
    else:
        x3 = x
    Bd, Td, Dd = x3.shape
    n = Bd * Td
    xf = x3.reshape(n, Dd)

    x_groups = x3.reshape(Bd, Td, NUM_GROUPS, GROUP_DIM)
    xn = jnp.stack(
        [jnp.sum(x_groups[:, :, i, :].reshape(-1, GROUP_DIM) ** 2, axis=1)
         for i in range(NUM_GROUPS)], axis=0)
    cbn = jnp.stack(
        [jnp.sum(codebooks[i] ** 2, axis=1) for i in range(NUM_GROUPS)],
        axis=0)
    cbt = jnp.transpose(codebooks, (0, 2, 1))

    rows = min(ROWS_PER_BLOCK, n)
    grid = (n // rows,)

    xq, idx, loss_sum = pl.pallas_call(
        _pq_body,
        grid=grid,
        in_specs=[
            pl.BlockSpec((rows, Dd), lambda i: (i, 0)),
            pl.BlockSpec((NUM_GROUPS, GROUP_DIM, CODES_PER_GROUP),
                         lambda i: (0, 0, 0)),
            pl.BlockSpec((NUM_GROUPS, CODES_PER_GROUP), lambda i: (0, 0)),
            pl.BlockSpec((NUM_GROUPS, rows), lambda i: (0, i)),
        ],
        out_specs=[
            pl.BlockSpec((rows, Dd), lambda i: (i, 0)),
            pl.BlockSpec((NUM_GROUPS, rows), lambda i: (0, i)),
            pl.BlockSpec((1, 1), lambda i: (0, 0)),
        ],
        out_shape=[
            jax.ShapeDtypeStruct((n, Dd), jnp.float32),
            jax.ShapeDtypeStruct((NUM_GROUPS, n), jnp.int32),
            jax.ShapeDtypeStruct((1, 1), jnp.float32),
        ],
    )(xf, cbt, cbn, xn)

    x_q = xq.reshape(original_shape)
    losses = loss_sum[0, 0] * (2.0 / (n * GROUP_DIM))
    indices = jnp.transpose(idx, (1, 0)).reshape(Bd, Td, NUM_GROUPS)
    if len(original_shape) == 2:
        indices = indices[:, 0, :]
    return (x_q, losses, indices)


# speedup vs baseline: 2.6340x; 1.0798x over previous
import functools

import jax
import jax.numpy as jnp
from jax.experimental import pallas as pl

B, T, INPUT_DIM = 16, 1024, 256
NUM_GROUPS = 8
CODES_PER_GROUP = 1024
GROUP_DIM = INPUT_DIM // NUM_GROUPS
N = B * T

ROWS_PER_BLOCK = 512


def _pq_body(x_ref, cbt_ref, cbn_ref, xn_ref, xq_ref, idx_ref, loss_ref):
    pid = pl.program_id(0)

    @pl.when(pid == 0)
    def _init():
        loss_ref[...] = jnp.zeros((1, 1), jnp.float32)

    acc = jnp.zeros((1, 1), jnp.float32)
    rows = x_ref.shape[0]
    iota_f = jax.lax.broadcasted_iota(
        jnp.int32, (rows, CODES_PER_GROUP), 1).astype(jnp.float32)
    for g in range(NUM_GROUPS):
        xg = x_ref[:, g * GROUP_DIM:(g + 1) * GROUP_DIM]
        mm = jnp.dot(xg, cbt_ref[g], preferred_element_type=jnp.float32)
        d = (xn_ref[g, :][:, None] + cbn_ref[g, :][None, :]) - 2.0 * mm
        m = jnp.min(d, axis=1, keepdims=True)
        idx_f = jnp.min(jnp.where(d == m, iota_f, float(CODES_PER_GROUP)),
                        axis=1, keepdims=True)
        idx_ref[g, :] = idx_f[:, 0].astype(jnp.int32)
        onehot = (iota_f == idx_f).astype(jnp.float32)
        xq = jax.lax.dot_general(
            onehot, cbt_ref[g], (((1,), (1,)), ((), ())),
            preferred_element_type=jnp.float32)
        xq_ref[:, g * GROUP_DIM:(g + 1) * GROUP_DIM] = xg + (xq - xg)
        acc = acc + jnp.sum(m, axis=0, keepdims=True)

    loss_ref[...] += acc


@jax.jit
def kernel(x, codebooks):
    original_shape = x.shape
    if x.ndim == 2:
        x3 = x[:, None, :]
    else:
        x3 = x
    Bd, Td, Dd = x3.shape
    n = Bd * Td
    xf = x3.reshape(n, Dd)

    x_groups = x3.reshape(Bd, Td, NUM_GROUPS, GROUP_DIM)
    xn = jnp.stack(
        [jnp.sum(x_groups[:, :, i, :].reshape(-1, GROUP_DIM) ** 2, axis=1)
         for i in range(NUM_GROUPS)], axis=0)
    cbn = jnp.stack(
        [jnp.sum(codebooks[i] ** 2, axis=1) for i in range(NUM_GROUPS)],
        axis=0)
    cbt = jnp.transpose(codebooks, (0, 2, 1))

    rows = min(ROWS_PER_BLOCK, n)
    grid = (n // rows,)

    xq, idx, loss_sum = pl.pallas_call(
        _pq_body,
        grid=grid,
        in_specs=[
            pl.BlockSpec((rows, Dd), lambda i: (i, 0)),
            pl.BlockSpec((NUM_GROUPS, GROUP_DIM, CODES_PER_GROUP),
                         lambda i: (0, 0, 0)),
            pl.BlockSpec((NUM_GROUPS, CODES_PER_GROUP), lambda i: (0, 0)),
            pl.BlockSpec((NUM_GROUPS, rows), lambda i: (0, i)),
        ],
        out_specs=[
            pl.BlockSpec((rows, Dd), lambda i: (i, 0)),
            pl.BlockSpec((NUM_GROUPS, rows), lambda i: (0, i)),
            pl.BlockSpec((1, 1), lambda i: (0, 0)),
        ],
        out_shape=[
            jax.ShapeDtypeStruct((n, Dd), jnp.float32),
            jax.ShapeDtypeStruct((NUM_GROUPS, n), jnp.int32),
            jax.ShapeDtypeStruct((1, 1), jnp.float32),
        ],
    )(xf, cbt, cbn, xn)

    x_q = xq.reshape(original_shape)
    losses = loss_sum[0, 0] * (2.0 / (n * GROUP_DIM))
    indices = jnp.transpose(idx, (1, 0)).reshape(Bd, Td, NUM_GROUPS)
    if len(original_shape) == 2:
        indices = indices[:, 0, :]
    return (x_q, losses, indices)


# speedup vs baseline: 2.6516x; 1.0067x over previous
import functools

import jax
import jax.numpy as jnp
from jax.experimental import pallas as pl

B, T, INPUT_DIM = 16, 1024, 256
NUM_GROUPS = 8
CODES_PER_GROUP = 1024
GROUP_DIM = INPUT_DIM // NUM_GROUPS
N = B * T

ROWS_PER_BLOCK = 512


def _pq_body(x_ref, cbt_ref, cbn_ref, xn_ref, xq_ref, idx_ref, loss_ref):
    pid = pl.program_id(0)

    @pl.when(pid == 0)
    def _init():
        loss_ref[...] = jnp.zeros((1, 1), jnp.float32)

    acc = jnp.zeros((1, 1), jnp.float32)
    rows = x_ref.shape[0]
    iota_f = jax.lax.broadcasted_iota(
        jnp.int32, (rows, CODES_PER_GROUP), 1).astype(jnp.float32)
    for g in range(NUM_GROUPS):
        xg = x_ref[:, g * GROUP_DIM:(g + 1) * GROUP_DIM]
        mm = jnp.dot(xg, cbt_ref[g], preferred_element_type=jnp.float32)
        d = (xn_ref[g, :][:, None] + cbn_ref[g, :][None, :]) - 2.0 * mm
        m = jnp.min(d, axis=1, keepdims=True)
        idx_f = jnp.min(jnp.where(d == m, iota_f, float(CODES_PER_GROUP)),
                        axis=1, keepdims=True)
        idx_ref[:, g:g + 1] = idx_f
        onehot = (iota_f == idx_f).astype(jnp.float32)
        xq = jax.lax.dot_general(
            onehot, cbt_ref[g], (((1,), (1,)), ((), ())),
            preferred_element_type=jnp.float32)
        xq_ref[:, g * GROUP_DIM:(g + 1) * GROUP_DIM] = xg + (xq - xg)
        acc = acc + jnp.sum(m, axis=0, keepdims=True)

    loss_ref[...] += acc


@jax.jit
def kernel(x, codebooks):
    original_shape = x.shape
    if x.ndim == 2:
        x3 = x[:, None, :]
    else:
        x3 = x
    Bd, Td, Dd = x3.shape
    n = Bd * Td
    xf = x3.reshape(n, Dd)

    x_groups = x3.reshape(Bd, Td, NUM_GROUPS, GROUP_DIM)
    xn = jnp.stack(
        [jnp.sum(x_groups[:, :, i, :].reshape(-1, GROUP_DIM) ** 2, axis=1)
         for i in range(NUM_GROUPS)], axis=0)
    cbn = jnp.stack(
        [jnp.sum(codebooks[i] ** 2, axis=1) for i in range(NUM_GROUPS)],
        axis=0)
    cbt = jnp.transpose(codebooks, (0, 2, 1))

    rows = min(ROWS_PER_BLOCK, n)
    grid = (n // rows,)

    xq, idx, loss_sum = pl.pallas_call(
        _pq_body,
        grid=grid,
        in_specs=[
            pl.BlockSpec((rows, Dd), lambda i: (i, 0)),
            pl.BlockSpec((NUM_GROUPS, GROUP_DIM, CODES_PER_GROUP),
                         lambda i: (0, 0, 0)),
            pl.BlockSpec((NUM_GROUPS, CODES_PER_GROUP), lambda i: (0, 0)),
            pl.BlockSpec((NUM_GROUPS, rows), lambda i: (0, i)),
        ],
        out_specs=[
            pl.BlockSpec((rows, Dd), lambda i: (i, 0)),
            pl.BlockSpec((rows, NUM_GROUPS), lambda i: (i, 0)),
            pl.BlockSpec((1, 1), lambda i: (0, 0)),
        ],
        out_shape=[
            jax.ShapeDtypeStruct((n, Dd), jnp.float32),
            jax.ShapeDtypeStruct((n, NUM_GROUPS), jnp.float32),
            jax.ShapeDtypeStruct((1, 1), jnp.float32),
        ],
    )(xf, cbt, cbn, xn)

    x_q = xq.reshape(original_shape)
    losses = loss_sum[0, 0] * (2.0 / (n * GROUP_DIM))
    indices = idx.astype(jnp.int32).reshape(Bd, Td, NUM_GROUPS)
    if len(original_shape) == 2:
        indices = indices[:, 0, :]
    return (x_q, losses, indices)


# speedup vs baseline: 2.7750x; 1.0465x over previous
import jax
import jax.numpy as jnp
from jax.experimental import pallas as pl

B, T, INPUT_DIM = 16, 1024, 256
NUM_GROUPS = 8
CODES_PER_GROUP = 1024
GROUP_DIM = INPUT_DIM // NUM_GROUPS
N = B * T

ROWS_PER_BLOCK = 512


def _pq_body(x_ref, cb_ref, cbn_ref, xq_ref, idx_ref, loss_ref):
    pid = pl.program_id(0)

    @pl.when(pid == 0)
    def _init():
        loss_ref[...] = jnp.zeros((1, 1), jnp.float32)

    acc = jnp.zeros((1, 1), jnp.float32)
    rows = x_ref.shape[0]
    iota_f = jax.lax.broadcasted_iota(
        jnp.int32, (rows, CODES_PER_GROUP), 1).astype(jnp.float32)
    for g in range(NUM_GROUPS):
        xg = x_ref[:, g * GROUP_DIM:(g + 1) * GROUP_DIM]
        xn = jnp.sum(xg * xg, axis=1, keepdims=True)
        mm = jax.lax.dot_general(
            xg, cb_ref[g], (((1,), (1,)), ((), ())),
            preferred_element_type=jnp.float32)
        d = (xn + cbn_ref[g, :][None, :]) - 2.0 * mm
        m = jnp.min(d, axis=1, keepdims=True)
        idx_f = jnp.min(jnp.where(d == m, iota_f, float(CODES_PER_GROUP)),
                        axis=1, keepdims=True)
        idx_ref[:, g:g + 1] = idx_f.astype(jnp.int32)
        onehot = (iota_f == idx_f).astype(jnp.float32)
        xq = jnp.dot(onehot, cb_ref[g],
                     preferred_element_type=jnp.float32)
        xq_ref[:, g * GROUP_DIM:(g + 1) * GROUP_DIM] = xg + (xq - xg)
        acc = acc + jnp.sum(m, axis=0, keepdims=True)

    loss_ref[...] += acc


@jax.jit
def kernel(x, codebooks):
    original_shape = x.shape
    if x.ndim == 2:
        x3 = x[:, None, :]
    else:
        x3 = x
    Bd, Td, Dd = x3.shape
    n = Bd * Td
    xf = x3.reshape(n, Dd)

    cbn = jnp.sum(codebooks**2, axis=2)

    rows = min(ROWS_PER_BLOCK, n)
    grid = (n // rows,)

    xq, idx, loss_sum = pl.pallas_call(
        _pq_body,
        grid=grid,
        in_specs=[
            pl.BlockSpec((rows, Dd), lambda i: (i, 0)),
            pl.BlockSpec((NUM_GROUPS, CODES_PER_GROUP, GROUP_DIM),
                         lambda i: (0, 0, 0)),
            pl.BlockSpec((NUM_GROUPS, CODES_PER_GROUP), lambda i: (0, 0)),
        ],
        out_specs=[
            pl.BlockSpec((rows, Dd), lambda i: (i, 0)),
            pl.BlockSpec((rows, NUM_GROUPS), lambda i: (i, 0)),
            pl.BlockSpec((1, 1), lambda i: (0, 0)),
        ],
        out_shape=[
            jax.ShapeDtypeStruct((n, Dd), jnp.float32),
            jax.ShapeDtypeStruct((n, NUM_GROUPS), jnp.int32),
            jax.ShapeDtypeStruct((1, 1), jnp.float32),
        ],
    )(xf, codebooks, cbn)

    x_q = xq.reshape(original_shape)
    losses = loss_sum[0, 0] * (2.0 / (n * GROUP_DIM))
    indices = idx.reshape(Bd, Td, NUM_GROUPS)
    if len(original_shape) == 2:
        indices = indices[:, 0, :]
    return (x_q, losses, indices)


# speedup vs baseline: 3.1274x; 1.1270x over previous
import functools

import jax
import jax.numpy as jnp
from jax import lax
from jax.experimental import pallas as pl
from jax.experimental.pallas import tpu as pltpu
from jax.experimental.pallas import tpu_sc as plsc

B, T, INPUT_DIM = 16, 1024, 256
NUM_GROUPS = 8
CODES_PER_GROUP = 1024
GROUP_DIM = INPUT_DIM // NUM_GROUPS
N = B * T

ROWS_PER_BLOCK = 512

_NC, _NS = 2, 16
_NW = _NC * _NS
_LOOKUPS = N * NUM_GROUPS
_PER_W = _LOOKUPS // _NW
_CHUNK = 1024


def _pq_body(x_ref, cb_ref, cbn_ref, idx_ref, loss_ref):
    pid = pl.program_id(0)

    @pl.when(pid == 0)
    def _init():
        loss_ref[...] = jnp.zeros((1, 1), jnp.float32)

    acc = jnp.zeros((1, 1), jnp.float32)
    rows = x_ref.shape[0]
    iota_f = jax.lax.broadcasted_iota(
        jnp.int32, (rows, CODES_PER_GROUP), 1).astype(jnp.float32)
    for g in range(NUM_GROUPS):
        xg = x_ref[:, g * GROUP_DIM:(g + 1) * GROUP_DIM]
        xn = jnp.sum(xg * xg, axis=1, keepdims=True)
        mm = jax.lax.dot_general(
            xg, cb_ref[g], (((1,), (1,)), ((), ())),
            preferred_element_type=jnp.float32)
        d = (xn + cbn_ref[g, :][None, :]) - 2.0 * mm
        m = jnp.min(d, axis=1, keepdims=True)
        idx_f = jnp.min(jnp.where(d == m, iota_f, float(CODES_PER_GROUP)),
                        axis=1, keepdims=True)
        idx_ref[:, g:g + 1] = (idx_f.astype(jnp.int32)
                               + g * CODES_PER_GROUP)
        acc = acc + jnp.sum(m, axis=0, keepdims=True)

    loss_ref[...] += acc


_sc_mesh = plsc.VectorSubcoreMesh(core_axis_name="c", subcore_axis_name="s")


@functools.partial(
    pl.kernel,
    mesh=_sc_mesh,
    compiler_params=pltpu.CompilerParams(use_tc_tiling_on_sc=False),
    out_type=jax.ShapeDtypeStruct((_LOOKUPS, GROUP_DIM), jnp.float32),
    scratch_types=[
        pltpu.VMEM((_CHUNK,), jnp.int32),
        pltpu.VMEM((_CHUNK, GROUP_DIM), jnp.float32),
        pltpu.SemaphoreType.DMA,
    ],
)
def _sc_gather(table_hbm, gidx_hbm, out_hbm, idx_v, rows_v, sem):
    wid = lax.axis_index("s") * _NC + lax.axis_index("c")
    base = wid * _PER_W
    for c in range(_PER_W // _CHUNK):
        off = base + c * _CHUNK
        pltpu.sync_copy(gidx_hbm.at[pl.ds(off, _CHUNK)], idx_v)
        pltpu.async_copy(table_hbm.at[idx_v], rows_v, sem).wait()
        pltpu.sync_copy(rows_v, out_hbm.at[pl.ds(off, _CHUNK)])


@jax.jit
def kernel(x, codebooks):
    original_shape = x.shape
    if x.ndim == 2:
        x3 = x[:, None, :]
    else:
        x3 = x
    Bd, Td, Dd = x3.shape
    n = Bd * Td
    xf = x3.reshape(n, Dd)

    cbn = jnp.sum(codebooks**2, axis=2)

    rows = min(ROWS_PER_BLOCK, n)
    grid = (n // rows,)

    gidx, loss_sum = pl.pallas_call(
        _pq_body,
        grid=grid,
        in_specs=[
            pl.BlockSpec((rows, Dd), lambda i: (i, 0)),
            pl.BlockSpec((NUM_GROUPS, CODES_PER_GROUP, GROUP_DIM),
                         lambda i: (0, 0, 0)),
            pl.BlockSpec((NUM_GROUPS, CODES_PER_GROUP), lambda i: (0, 0)),
        ],
        out_specs=[
            pl.BlockSpec((rows, NUM_GROUPS), lambda i: (i, 0)),
            pl.BlockSpec((1, 1), lambda i: (0, 0)),
        ],
        out_shape=[
            jax.ShapeDtypeStruct((n, NUM_GROUPS), jnp.int32),
            jax.ShapeDtypeStruct((1, 1), jnp.float32),
        ],
    )(xf, codebooks, cbn)

    table = codebooks.reshape(NUM_GROUPS * CODES_PER_GROUP, GROUP_DIM)
    xqflat = _sc_gather(table, gidx.reshape(-1))

    x_q = xqflat.reshape(original_shape)
    losses = loss_sum[0, 0] * (2.0 / (n * GROUP_DIM))
    offs = jnp.arange(NUM_GROUPS, dtype=jnp.int32) * CODES_PER_GROUP
    indices = (gidx - offs[None, :]).reshape(Bd, Td, NUM_GROUPS)
    if len(original_shape) == 2:
        indices = indices[:, 0, :]
    return (x_q, losses, indices)


# speedup vs baseline: 3.5053x; 1.1208x over previous
import functools

import jax
import jax.numpy as jnp
from jax import lax
from jax.experimental import pallas as pl
from jax.experimental.pallas import tpu as pltpu
from jax.experimental.pallas import tpu_sc as plsc

B, T, INPUT_DIM = 16, 1024, 256
NUM_GROUPS = 8
CODES_PER_GROUP = 1024
GROUP_DIM = INPUT_DIM // NUM_GROUPS
N = B * T

ROWS_PER_BLOCK = 1024

_NC, _NS = 2, 16
_NW = _NC * _NS
_LOOKUPS = N * NUM_GROUPS
_PER_W = _LOOKUPS // _NW
_CHUNK = 1024


def _pq_body(x_ref, cb_ref, cbn_ref, idx_ref, loss_ref):
    pid = pl.program_id(0)

    @pl.when(pid == 0)
    def _init():
        loss_ref[...] = jnp.zeros((1, 1), jnp.float32)

    acc = jnp.zeros((1, 1), jnp.float32)
    rows = x_ref.shape[0]
    iota_f = jax.lax.broadcasted_iota(
        jnp.int32, (rows, CODES_PER_GROUP), 1).astype(jnp.float32)
    for g in range(NUM_GROUPS):
        xg = x_ref[:, g * GROUP_DIM:(g + 1) * GROUP_DIM]
        xn = jnp.sum(xg * xg, axis=1, keepdims=True)
        mm = jax.lax.dot_general(
            xg, cb_ref[g], (((1,), (1,)), ((), ())),
            preferred_element_type=jnp.float32)
        d = (xn + cbn_ref[g, :][None, :]) - 2.0 * mm
        m = jnp.min(d, axis=1, keepdims=True)
        idx_f = jnp.min(jnp.where(d == m, iota_f, float(CODES_PER_GROUP)),
                        axis=1, keepdims=True)
        idx_ref[:, g:g + 1] = (idx_f.astype(jnp.int32)
                               + g * CODES_PER_GROUP)
        acc = acc + jnp.sum(m, axis=0, keepdims=True)

    loss_ref[...] += acc


_sc_mesh = plsc.VectorSubcoreMesh(core_axis_name="c", subcore_axis_name="s")


@functools.partial(
    pl.kernel,
    mesh=_sc_mesh,
    compiler_params=pltpu.CompilerParams(use_tc_tiling_on_sc=False),
    out_type=jax.ShapeDtypeStruct((_LOOKUPS, GROUP_DIM), jnp.float32),
    scratch_types=[
        pltpu.VMEM((_CHUNK,), jnp.int32),
        pltpu.VMEM((_CHUNK, GROUP_DIM), jnp.float32),
        pltpu.SemaphoreType.DMA,
    ],
)
def _sc_gather(table_hbm, gidx_hbm, out_hbm, idx_v, rows_v, sem):
    wid = lax.axis_index("s") * _NC + lax.axis_index("c")
    base = wid * _PER_W
    for c in range(_PER_W // _CHUNK):
        off = base + c * _CHUNK
        pltpu.sync_copy(gidx_hbm.at[pl.ds(off, _CHUNK)], idx_v)
        pltpu.async_copy(table_hbm.at[idx_v], rows_v, sem).wait()
        pltpu.sync_copy(rows_v, out_hbm.at[pl.ds(off, _CHUNK)])


@jax.jit
def kernel(x, codebooks):
    original_shape = x.shape
    if x.ndim == 2:
        x3 = x[:, None, :]
    else:
        x3 = x
    Bd, Td, Dd = x3.shape
    n = Bd * Td
    xf = x3.reshape(n, Dd)

    cbn = jnp.sum(codebooks**2, axis=2)

    rows = min(ROWS_PER_BLOCK, n)
    grid = (n // rows,)

    gidx, loss_sum = pl.pallas_call(
        _pq_body,
        grid=grid,
        in_specs=[
            pl.BlockSpec((rows, Dd), lambda i: (i, 0)),
            pl.BlockSpec((NUM_GROUPS, CODES_PER_GROUP, GROUP_DIM),
                         lambda i: (0, 0, 0)),
            pl.BlockSpec((NUM_GROUPS, CODES_PER_GROUP), lambda i: (0, 0)),
        ],
        out_specs=[
            pl.BlockSpec((rows, NUM_GROUPS), lambda i: (i, 0)),
            pl.BlockSpec((1, 1), lambda i: (0, 0)),
        ],
        out_shape=[
            jax.ShapeDtypeStruct((n, NUM_GROUPS), jnp.int32),
            jax.ShapeDtypeStruct((1, 1), jnp.float32),
        ],
    )(xf, codebooks, cbn)

    table = codebooks.reshape(NUM_GROUPS * CODES_PER_GROUP, GROUP_DIM)
    xqflat = _sc_gather(table, gidx.reshape(-1))

    x_q = xqflat.reshape(original_shape)
    losses = loss_sum[0, 0] * (2.0 / (n * GROUP_DIM))
    offs = jnp.arange(NUM_GROUPS, dtype=jnp.int32) * CODES_PER_GROUP
    indices = (gidx - offs[None, :]).reshape(Bd, Td, NUM_GROUPS)
    if len(original_shape) == 2:
        indices = indices[:, 0, :]
    return (x_q, losses, indices)


# speedup vs baseline: 3.5759x; 1.0201x over previous
import functools

import jax
import jax.numpy as jnp
from jax import lax
from jax.experimental import pallas as pl
from jax.experimental.pallas import tpu as pltpu
from jax.experimental.pallas import tpu_sc as plsc

B, T, INPUT_DIM = 16, 1024, 256
NUM_GROUPS = 8
CODES_PER_GROUP = 1024
GROUP_DIM = INPUT_DIM // NUM_GROUPS
N = B * T

ROWS_PER_BLOCK = 1024

_NC, _NS = 2, 16
_NW = _NC * _NS
_LOOKUPS = N * NUM_GROUPS
_PER_W = _LOOKUPS // _NW
_CHUNK = 1024


def _pq_body(x_ref, cb2_ref, cbn_ref, idx_ref, loss_ref):
    pid = pl.program_id(0)

    @pl.when(pid == 0)
    def _init():
        loss_ref[...] = jnp.zeros((1, 1), jnp.float32)

    acc = jnp.zeros((1, 1), jnp.float32)
    rows = x_ref.shape[0]
    iota_f = jax.lax.broadcasted_iota(
        jnp.int32, (rows, CODES_PER_GROUP), 1).astype(jnp.float32)
    for g in range(NUM_GROUPS):
        xg = x_ref[:, g * GROUP_DIM:(g + 1) * GROUP_DIM]
        xn = jnp.sum(xg * xg, axis=1, keepdims=True)
        mm2 = jax.lax.dot_general(
            xg, cb2_ref[g], (((1,), (1,)), ((), ())),
            preferred_element_type=jnp.float32)
        d = (xn + cbn_ref[g, :][None, :]) - mm2
        m = jnp.min(d, axis=1, keepdims=True)
        idx_f = jnp.min(jnp.where(d == m, iota_f, float(CODES_PER_GROUP)),
                        axis=1, keepdims=True)
        idx_ref[:, g:g + 1] = (idx_f.astype(jnp.int32)
                               + g * CODES_PER_GROUP)
        acc = acc + jnp.sum(m, axis=0, keepdims=True)

    loss_ref[...] += acc


_sc_mesh = plsc.VectorSubcoreMesh(core_axis_name="c", subcore_axis_name="s")


@functools.partial(
    pl.kernel,
    mesh=_sc_mesh,
    compiler_params=pltpu.CompilerParams(use_tc_tiling_on_sc=False),
    out_type=jax.ShapeDtypeStruct((_LOOKUPS, GROUP_DIM), jnp.float32),
    scratch_types=[
        pltpu.VMEM((2, _CHUNK), jnp.int32),
        pltpu.VMEM((2, _CHUNK, GROUP_DIM), jnp.float32),
        pltpu.SemaphoreType.DMA,
        pltpu.SemaphoreType.DMA,
    ],
)
def _sc_gather(table_hbm, gidx_hbm, out_hbm, idx_v, rows_v, sem0, sem1):
    wid = lax.axis_index("s") * _NC + lax.axis_index("c")
    base = wid * _PER_W
    n_chunks = _PER_W // _CHUNK
    sems = (sem0, sem1)
    pltpu.sync_copy(gidx_hbm.at[pl.ds(base, _CHUNK)], idx_v.at[0])
    gathers = [pltpu.async_copy(table_hbm.at[idx_v.at[0]], rows_v.at[0],
                                sems[0]), None]
    for c in range(n_chunks):
        cur = c % 2
        if c + 1 < n_chunks:
            nxt = (c + 1) % 2
            off = base + (c + 1) * _CHUNK
            pltpu.sync_copy(gidx_hbm.at[pl.ds(off, _CHUNK)], idx_v.at[nxt])
            gathers[nxt] = pltpu.async_copy(
                table_hbm.at[idx_v.at[nxt]], rows_v.at[nxt], sems[nxt])
        gathers[cur].wait()
        pltpu.sync_copy(rows_v.at[cur],
                        out_hbm.at[pl.ds(base + c * _CHUNK, _CHUNK)])


@jax.jit
def kernel(x, codebooks):
    original_shape = x.shape
    if x.ndim == 2:
        x3 = x[:, None, :]
    else:
        x3 = x
    Bd, Td, Dd = x3.shape
    n = Bd * Td
    xf = x3.reshape(n, Dd)

    cbn = jnp.sum(codebooks**2, axis=2)
    cb2 = codebooks * 2.0

    rows = min(ROWS_PER_BLOCK, n)
    grid = (n // rows,)

    gidx, loss_sum = pl.pallas_call(
        _pq_body,
        grid=grid,
        in_specs=[
            pl.BlockSpec((rows, Dd), lambda i: (i, 0)),
            pl.BlockSpec((NUM_GROUPS, CODES_PER_GROUP, GROUP_DIM),
                         lambda i: (0, 0, 0)),
            pl.BlockSpec((NUM_GROUPS, CODES_PER_GROUP), lambda i: (0, 0)),
        ],
        out_specs=[
            pl.BlockSpec((rows, NUM_GROUPS), lambda i: (i, 0)),
            pl.BlockSpec((1, 1), lambda i: (0, 0)),
        ],
        out_shape=[
            jax.ShapeDtypeStruct((n, NUM_GROUPS), jnp.int32),
            jax.ShapeDtypeStruct((1, 1), jnp.float32),
        ],
    )(xf, cb2, cbn)

    table = codebooks.reshape(NUM_GROUPS * CODES_PER_GROUP, GROUP_DIM)
    xqflat = _sc_gather(table, gidx.reshape(-1))

    x_q = xqflat.reshape(original_shape)
    losses = loss_sum[0, 0] * (2.0 / (n * GROUP_DIM))
    offs = jnp.arange(NUM_GROUPS, dtype=jnp.int32) * CODES_PER_GROUP
    indices = (gidx - offs[None, :]).reshape(Bd, Td, NUM_GROUPS)
    if len(original_shape) == 2:
        indices = indices[:, 0, :]
    return (x_q, losses, indices)


# speedup vs baseline: 3.6716x; 1.0268x over previous
import functools

import jax
import jax.numpy as jnp
from jax import lax
from jax.experimental import pallas as pl
from jax.experimental.pallas import tpu as pltpu
from jax.experimental.pallas import tpu_sc as plsc

B, T, INPUT_DIM = 16, 1024, 256
NUM_GROUPS = 8
CODES_PER_GROUP = 1024
GROUP_DIM = INPUT_DIM // NUM_GROUPS
N = B * T

ROWS_PER_BLOCK = 2048

_NC, _NS = 2, 16
_NW = _NC * _NS
_LOOKUPS = N * NUM_GROUPS
_PER_W = _LOOKUPS // _NW
_CHUNK = 1024


def _pq_body(x_ref, cb2_ref, cbn_ref, idx_ref, loss_ref):
    pid = pl.program_id(0)

    @pl.when(pid == 0)
    def _init():
        loss_ref[...] = jnp.zeros((1, 1), jnp.float32)

    acc = jnp.zeros((1, 1), jnp.float32)
    rows = x_ref.shape[0]
    iota_f = jax.lax.broadcasted_iota(
        jnp.int32, (rows, CODES_PER_GROUP), 1).astype(jnp.float32)
    for g in range(NUM_GROUPS):
        xg = x_ref[:, g * GROUP_DIM:(g + 1) * GROUP_DIM]
        xn = jnp.sum(xg * xg, axis=1, keepdims=True)
        mm2 = jax.lax.dot_general(
            xg, cb2_ref[g], (((1,), (1,)), ((), ())),
            preferred_element_type=jnp.float32)
        d = (xn + cbn_ref[g, :][None, :]) - mm2
        m = jnp.min(d, axis=1, keepdims=True)
        idx_f = jnp.min(jnp.where(d == m, iota_f, float(CODES_PER_GROUP)),
                        axis=1, keepdims=True)
        idx_ref[:, g:g + 1] = (idx_f.astype(jnp.int32)
                               + g * CODES_PER_GROUP)
        acc = acc + jnp.sum(m, axis=0, keepdims=True)

    loss_ref[...] += acc


_sc_mesh = plsc.VectorSubcoreMesh(core_axis_name="c", subcore_axis_name="s")


@functools.partial(
    pl.kernel,
    mesh=_sc_mesh,
    compiler_params=pltpu.CompilerParams(use_tc_tiling_on_sc=False),
    out_type=jax.ShapeDtypeStruct((_LOOKUPS, GROUP_DIM), jnp.float32),
    scratch_types=[
        pltpu.VMEM((2, _CHUNK), jnp.int32),
        pltpu.VMEM((2, _CHUNK, GROUP_DIM), jnp.float32),
        pltpu.SemaphoreType.DMA,
        pltpu.SemaphoreType.DMA,
    ],
)
def _sc_gather(table_hbm, gidx_hbm, out_hbm, idx_v, rows_v, sem0, sem1):
    wid = lax.axis_index("s") * _NC + lax.axis_index("c")
    base = wid * _PER_W
    n_chunks = _PER_W // _CHUNK
    sems = (sem0, sem1)
    pltpu.sync_copy(gidx_hbm.at[pl.ds(base, _CHUNK)], idx_v.at[0])
    gathers = [pltpu.async_copy(table_hbm.at[idx_v.at[0]], rows_v.at[0],
                                sems[0]), None]
    for c in range(n_chunks):
        cur = c % 2
        if c + 1 < n_chunks:
            nxt = (c + 1) % 2
            off = base + (c + 1) * _CHUNK
            pltpu.sync_copy(gidx_hbm.at[pl.ds(off, _CHUNK)], idx_v.at[nxt])
            gathers[nxt] = pltpu.async_copy(
                table_hbm.at[idx_v.at[nxt]], rows_v.at[nxt], sems[nxt])
        gathers[cur].wait()
        pltpu.sync_copy(rows_v.at[cur],
                        out_hbm.at[pl.ds(base + c * _CHUNK, _CHUNK)])


@jax.jit
def kernel(x, codebooks):
    original_shape = x.shape
    if x.ndim == 2:
        x3 = x[:, None, :]
    else:
        x3 = x
    Bd, Td, Dd = x3.shape
    n = Bd * Td
    xf = x3.reshape(n, Dd)

    cbn = jnp.sum(codebooks**2, axis=2)
    cb2 = codebooks * 2.0

    rows = min(ROWS_PER_BLOCK, n)
    grid = (n // rows,)

    gidx, loss_sum = pl.pallas_call(
        _pq_body,
        grid=grid,
        in_specs=[
            pl.BlockSpec((rows, Dd), lambda i: (i, 0)),
            pl.BlockSpec((NUM_GROUPS, CODES_PER_GROUP, GROUP_DIM),
                         lambda i: (0, 0, 0)),
            pl.BlockSpec((NUM_GROUPS, CODES_PER_GROUP), lambda i: (0, 0)),
        ],
        out_specs=[
            pl.BlockSpec((rows, NUM_GROUPS), lambda i: (i, 0)),
            pl.BlockSpec((1, 1), lambda i: (0, 0)),
        ],
        out_shape=[
            jax.ShapeDtypeStruct((n, NUM_GROUPS), jnp.int32),
            jax.ShapeDtypeStruct((1, 1), jnp.float32),
        ],
    )(xf, cb2, cbn)

    table = codebooks.reshape(NUM_GROUPS * CODES_PER_GROUP, GROUP_DIM)
    xqflat = _sc_gather(table, gidx.reshape(-1))

    x_q = xqflat.reshape(original_shape)
    losses = loss_sum[0, 0] * (2.0 / (n * GROUP_DIM))
    offs = jnp.arange(NUM_GROUPS, dtype=jnp.int32) * CODES_PER_GROUP
    indices = (gidx - offs[None, :]).reshape(Bd, Td, NUM_GROUPS)
    if len(original_shape) == 2:
        indices = indices[:, 0, :]
    return (x_q, losses, indices)
